# baseline (speedup 1.0000x reference)
import functools

import jax
import jax.numpy as jnp
from jax import lax
from jax.experimental import pallas as pl
from jax.experimental.pallas import tpu as pltpu
from jax.experimental.pallas import tpu_sc as plsc

B = 16384
V = 1000
D_V = 16
D_J = 8
D_O = D_V + D_J

_INFO = plsc.get_sparse_core_info()
_NC, _NS, _L = _INFO.num_cores, _INFO.num_subcores, _INFO.num_lanes
_NW = _NC * _NS
_BPW = B // _NW
_GRP = _BPW // _L


def _sc_body(idxv_hbm, idxj_hbm, wvt_hbm, wjt_hbm, out_hbm,
             idxv_v, idxj_v, wv_v, wj_v, comb_v, sem_v, sem_j, sem_i, sem_o):
    wid = lax.axis_index("s") * _NC + lax.axis_index("c")
    base = wid * _BPW

    cps = [pltpu.async_copy(idxv_hbm.at[pl.ds(base, _BPW)], idxv_v, sem_i),
           pltpu.async_copy(idxj_hbm.at[pl.ds(base, _BPW)], idxj_v, sem_i),
           pltpu.async_copy(wvt_hbm, wv_v, sem_v),
           pltpu.async_copy(wjt_hbm, wj_v, sem_j)]
    for cp in cps:
        cp.wait()

    cvecs = [jnp.broadcast_to(jnp.int32(c), (_L,)) for c in range(D_V)]

    @plsc.parallel_loop(0, _GRP, unroll=2)
    def _group(g):
        off = g * _L
        ivv = idxv_v[pl.ds(off, _L)]
        ivj = idxj_v[pl.ds(off, _L)]
        for c in range(D_V):
            comb_v[c, pl.ds(off, _L)] = plsc.load_gather(wv_v, [cvecs[c], ivv])
        for c in range(D_J):
            comb_v[D_V + c, pl.ds(off, _L)] = plsc.load_gather(
                wj_v, [cvecs[c], ivj])

    pltpu.sync_copy(comb_v, out_hbm.at[:, pl.ds(base, _BPW)])


@jax.jit
def _gene_encode(idxv, idxj, wvt, wjt):
    mesh = plsc.VectorSubcoreMesh(core_axis_name="c", subcore_axis_name="s")
    k = functools.partial(
        pl.kernel,
        mesh=mesh,
        compiler_params=pltpu.CompilerParams(
            needs_layout_passes=False, disable_bounds_checks=True),
        out_type=jax.ShapeDtypeStruct((D_O, B), jnp.float32),
        scratch_types=[
            pltpu.VMEM((_BPW,), jnp.int32),
            pltpu.VMEM((_BPW,), jnp.int32),
            pltpu.VMEM((D_V, V), jnp.float32),
            pltpu.VMEM((D_J, V), jnp.float32),
            pltpu.VMEM((D_O, _BPW), jnp.float32),
            pltpu.SemaphoreType.DMA,
            pltpu.SemaphoreType.DMA,
            pltpu.SemaphoreType.DMA,
            pltpu.SemaphoreType.DMA,
        ],
    )(_sc_body)
    return k(idxv, idxj, wvt, wjt)


def kernel(TRA_v_gene, TRA_j_gene, W_v, W_j):
    zt = _gene_encode(
        TRA_v_gene.astype(jnp.int32),
        TRA_j_gene.astype(jnp.int32),
        W_v.T,
        W_j.T,
    )
    return zt.T


# speedup vs baseline: 1.0003x; 1.0003x over previous
import functools

import jax
import jax.numpy as jnp
from jax import lax
from jax.experimental import pallas as pl
from jax.experimental.pallas import tpu as pltpu
from jax.experimental.pallas import tpu_sc as plsc

B = 16384
V = 1000
D_V = 16
D_J = 8
D_O = D_V + D_J

_INFO = plsc.get_sparse_core_info()
_NC, _NS, _L = _INFO.num_cores, _INFO.num_subcores, _INFO.num_lanes
_NW = _NC * _NS
_BPW = B // _NW
_GRP = _BPW // _L


def _sc_body(idxv_hbm, idxj_hbm, wvt_hbm, wjt_hbm, out_hbm,
             idxv_v, idxj_v, wv_v, wj_v, comb_v, sem_v, sem_j, sem_i, sem_o):
    wid = lax.axis_index("s") * _NC + lax.axis_index("c")
    base = wid * _BPW

    cp_wv = pltpu.async_copy(wvt_hbm, wv_v, sem_v)
    cp_wj = pltpu.async_copy(wjt_hbm, wj_v, sem_j)
    cp_i1 = pltpu.async_copy(idxv_hbm.at[pl.ds(base, _BPW)], idxv_v, sem_i)
    cp_i2 = pltpu.async_copy(idxj_hbm.at[pl.ds(base, _BPW)], idxj_v, sem_i)

    cvecs = [jnp.broadcast_to(jnp.int32(c), (_L,)) for c in range(D_V)]

    cp_i1.wait()
    cp_i2.wait()
    cp_wv.wait()

    @plsc.parallel_loop(0, _GRP, unroll=2)
    def _vgroup(g):
        off = g * _L
        ivv = idxv_v[pl.ds(off, _L)]
        for c in range(D_V):
            comb_v[c, pl.ds(off, _L)] = plsc.load_gather(wv_v, [cvecs[c], ivv])

    cp_wj.wait()

    @plsc.parallel_loop(0, _GRP, unroll=2)
    def _jgroup(g):
        off = g * _L
        ivj = idxj_v[pl.ds(off, _L)]
        for c in range(D_J):
            comb_v[D_V + c, pl.ds(off, _L)] = plsc.load_gather(
                wj_v, [cvecs[c], ivj])

    pltpu.sync_copy(comb_v, out_hbm.at[:, pl.ds(base, _BPW)])


@jax.jit
def _gene_encode(idxv, idxj, wvt, wjt):
    mesh = plsc.VectorSubcoreMesh(core_axis_name="c", subcore_axis_name="s")
    k = functools.partial(
        pl.kernel,
        mesh=mesh,
        compiler_params=pltpu.CompilerParams(
            needs_layout_passes=False, disable_bounds_checks=True),
        out_type=jax.ShapeDtypeStruct((D_O, B), jnp.float32),
        scratch_types=[
            pltpu.VMEM((_BPW,), jnp.int32),
            pltpu.VMEM((_BPW,), jnp.int32),
            pltpu.VMEM((D_V, V), jnp.float32),
            pltpu.VMEM((D_J, V), jnp.float32),
            pltpu.VMEM((D_O, _BPW), jnp.float32),
            pltpu.SemaphoreType.DMA,
            pltpu.SemaphoreType.DMA,
            pltpu.SemaphoreType.DMA,
            pltpu.SemaphoreType.DMA,
        ],
    )(_sc_body)
    return k(idxv, idxj, wvt, wjt)


def kernel(TRA_v_gene, TRA_j_gene, W_v, W_j):
    zt = _gene_encode(
        TRA_v_gene.astype(jnp.int32),
        TRA_j_gene.astype(jnp.int32),
        W_v.T,
        W_j.T,
    )
    return zt.T


# speedup vs baseline: 1.0857x; 1.0854x over previous
import functools

import jax
import jax.numpy as jnp
from jax import lax
from jax.experimental import pallas as pl
from jax.experimental.pallas import tpu as pltpu
from jax.experimental.pallas import tpu_sc as plsc

B = 16384
V = 1000
D_V = 16
D_J = 8
D_O = D_V + D_J

_INFO = plsc.get_sparse_core_info()
_NC, _NS, _L = _INFO.num_cores, _INFO.num_subcores, _INFO.num_lanes
_NW = _NC * _NS
_BPW = B // _NW
_GRP = _BPW // _L


def _sc_body(idxv_hbm, idxj_hbm, wvt_hbm, wjt_hbm, out_hbm,
             idxv_v, idxj_v, wv_v, wj_v, comb_v, wv_s, wj_s,
             sem_v, sem_j, sem_i, sem_o):
    sid = lax.axis_index("s")
    wid = sid * _NC + lax.axis_index("c")
    base = wid * _BPW

    @pl.when(sid == 0)
    def _fetch():
        pltpu.sync_copy(wvt_hbm, wv_s)
        pltpu.sync_copy(wjt_hbm, wj_s)

    cp_i1 = pltpu.async_copy(idxv_hbm.at[pl.ds(base, _BPW)], idxv_v, sem_i)
    cp_i2 = pltpu.async_copy(idxj_hbm.at[pl.ds(base, _BPW)], idxj_v, sem_i)

    plsc.subcore_barrier()
    cp_wv = pltpu.async_copy(wv_s, wv_v, sem_v)
    cp_wj = pltpu.async_copy(wj_s, wj_v, sem_j)

    cvecs = [jnp.broadcast_to(jnp.int32(c), (_L,)) for c in range(D_V)]

    cp_i1.wait()
    cp_i2.wait()
    cp_wv.wait()

    @plsc.parallel_loop(0, _GRP, unroll=2)
    def _vgroup(g):
        off = g * _L
        ivv = idxv_v[pl.ds(off, _L)]
        for c in range(D_V):
            comb_v[c, pl.ds(off, _L)] = plsc.load_gather(wv_v, [cvecs[c], ivv])

    cp_wj.wait()

    @plsc.parallel_loop(0, _GRP, unroll=2)
    def _jgroup(g):
        off = g * _L
        ivj = idxj_v[pl.ds(off, _L)]
        for c in range(D_J):
            comb_v[D_V + c, pl.ds(off, _L)] = plsc.load_gather(
                wj_v, [cvecs[c], ivj])

    pltpu.sync_copy(comb_v, out_hbm.at[:, pl.ds(base, _BPW)])


@jax.jit
def _gene_encode(idxv, idxj, wvt, wjt):
    mesh = plsc.VectorSubcoreMesh(core_axis_name="c", subcore_axis_name="s")
    k = functools.partial(
        pl.kernel,
        mesh=mesh,
        compiler_params=pltpu.CompilerParams(
            needs_layout_passes=False, disable_bounds_checks=True),
        out_type=jax.ShapeDtypeStruct((D_O, B), jnp.float32),
        scratch_types=[
            pltpu.VMEM((_BPW,), jnp.int32),
            pltpu.VMEM((_BPW,), jnp.int32),
            pltpu.VMEM((D_V, V), jnp.float32),
            pltpu.VMEM((D_J, V), jnp.float32),
            pltpu.VMEM((D_O, _BPW), jnp.float32),
            pltpu.VMEM_SHARED((D_V, V), jnp.float32),
            pltpu.VMEM_SHARED((D_J, V), jnp.float32),
            pltpu.SemaphoreType.DMA,
            pltpu.SemaphoreType.DMA,
            pltpu.SemaphoreType.DMA,
            pltpu.SemaphoreType.DMA,
        ],
    )(_sc_body)
    return k(idxv, idxj, wvt, wjt)


def kernel(TRA_v_gene, TRA_j_gene, W_v, W_j):
    zt = _gene_encode(
        TRA_v_gene.astype(jnp.int32),
        TRA_j_gene.astype(jnp.int32),
        W_v.T,
        W_j.T,
    )
    return zt.T
